# baseline (speedup 1.0000x reference)
import functools

import jax
import jax.numpy as jnp
from jax import lax
from jax.experimental import pallas as pl
from jax.experimental.pallas import tpu as pltpu
from jax.experimental.pallas import tpu_sc as plsc

N_NODES = 10000
N_EDGES = 320000
NC, NS, L = 2, 16, 16
NW = NC * NS
NPAD = 10240
NBLK = N_EDGES // 128
BPT = NBLK // NW
REM = NBLK - BPT * NW
EMAX = (BPT + 1) * 128


def _edge_pass():
    mesh = plsc.VectorSubcoreMesh(core_axis_name="c", subcore_axis_name="s")

    @functools.partial(
        pl.kernel,
        mesh=mesh,
        compiler_params=pltpu.CompilerParams(needs_layout_passes=False),
        out_type=jax.ShapeDtypeStruct((2, NW, NPAD), jnp.float32),
        scratch_types=[
            pltpu.VMEM((N_NODES,), jnp.float32),
            pltpu.VMEM((N_NODES,), jnp.float32),
            pltpu.VMEM((N_NODES,), jnp.float32),
            pltpu.VMEM((2, EMAX), jnp.int32),
            pltpu.VMEM((NPAD,), jnp.float32),
            pltpu.VMEM((NPAD,), jnp.float32),
            pltpu.SemaphoreType.DMA,
        ],
    )
    def edge_pass(px_hbm, py_hbm, pz_hbm, ei_hbm, out_hbm,
                  px_v, py_v, pz_v, rc_v, s_v, d_v, sem):
        wid = lax.axis_index("s") * NC + lax.axis_index("c")
        extra = jnp.where(wid < REM, 1, 0)
        base = (BPT * wid + jnp.minimum(wid, REM)) * 128
        cp_px = pltpu.async_copy(px_hbm, px_v, sem)
        cp_py = pltpu.async_copy(py_hbm, py_v, sem)
        cp_pz = pltpu.async_copy(pz_hbm, pz_v, sem)

        @pl.when(wid < REM)
        def _():
            pltpu.sync_copy(ei_hbm.at[:, pl.ds(base, (BPT + 1) * 128)], rc_v)

        @pl.when(wid >= REM)
        def _():
            pltpu.sync_copy(ei_hbm.at[:, pl.ds(base, BPT * 128)],
                            rc_v.at[:, pl.ds(0, BPT * 128)])

        zf = jnp.zeros((L,), jnp.float32)

        @plsc.parallel_loop(0, NPAD // L, 1, unroll=8)
        def _(i):
            s_v[pl.ds(i * L, L)] = zf
            d_v[pl.ds(i * L, L)] = zf

        cp_px.wait()
        cp_py.wait()
        cp_pz.wait()
        ones = jnp.ones((L,), jnp.float32)

        @plsc.parallel_loop(0, (BPT + extra) * (128 // L), 1, unroll=8)
        def _(i):
            r = rc_v[0, pl.ds(i * L, L)]
            c = rc_v[1, pl.ds(i * L, L)]
            m = r != c
            dx = plsc.load_gather(px_v, [r]) - plsc.load_gather(px_v, [c])
            dy = plsc.load_gather(py_v, [r]) - plsc.load_gather(py_v, [c])
            dz = plsc.load_gather(pz_v, [r]) - plsc.load_gather(pz_v, [c])
            dsq = jnp.maximum(dx * dx + dy * dy + dz * dz, 1e-12)
            yb = jnp.int32(0x5F3759DF) - (plsc.bitcast(dsq, jnp.int32) >> 1)
            y = plsc.bitcast(yb, jnp.float32)
            hx = 0.5 * dsq
            y = y * (1.5 - hx * y * y)
            y = y * (1.5 - hx * y * y)
            plsc.addupdate_scatter(s_v, [r], y, mask=m)
            plsc.addupdate_scatter(d_v, [r], ones, mask=m)

        pltpu.sync_copy(s_v, out_hbm.at[0, wid])
        pltpu.sync_copy(d_v, out_hbm.at[1, wid])

    return edge_pass


_BN = 2048


def _mm_body(x_ref, w_ref, o_ref):
    o_ref[...] = jnp.dot(x_ref[...], w_ref[...],
                         preferred_element_type=jnp.float32)


def _scale_body(a_ref, p_ref, o_ref):
    s = jnp.sum(p_ref[0], axis=0)
    deg = jnp.sum(p_ref[1], axis=0)
    scale = (1.0 + s) / (deg + 1.0 + 1e-16)
    o_ref[...] = a_ref[...] * scale[:, None]


@jax.jit
def kernel(x, pos, edge_index, weight, attention):
    del attention
    partials = _edge_pass()(pos[:, 0], pos[:, 1], pos[:, 2], edge_index)

    acc = pl.pallas_call(
        _mm_body,
        grid=(NPAD // _BN,),
        in_specs=[
            pl.BlockSpec((_BN, 128), lambda i: (i, 0)),
            pl.BlockSpec((128, 128), lambda i: (0, 0)),
        ],
        out_specs=pl.BlockSpec((_BN, 128), lambda i: (i, 0)),
        out_shape=jax.ShapeDtypeStruct((N_NODES, 128), jnp.float32),
    )(x, weight[0])
    return pl.pallas_call(
        _scale_body,
        grid=(NPAD // _BN,),
        in_specs=[
            pl.BlockSpec((_BN, 128), lambda i: (i, 0)),
            pl.BlockSpec((2, NW, _BN), lambda i: (0, 0, i)),
        ],
        out_specs=pl.BlockSpec((_BN, 128), lambda i: (i, 0)),
        out_shape=jax.ShapeDtypeStruct((N_NODES, 128), jnp.float32),
    )(acc, partials)


# speedup vs baseline: 1.0184x; 1.0184x over previous
import functools

import jax
import jax.numpy as jnp
from jax import lax
from jax.experimental import pallas as pl
from jax.experimental.pallas import tpu as pltpu
from jax.experimental.pallas import tpu_sc as plsc

N_NODES = 10000
N_EDGES = 320000
NC, NS, L = 2, 16, 16
NW = NC * NS
NPAD = 10240
NBLK = N_EDGES // 128
BPT = NBLK // NW
REM = NBLK - BPT * NW
EMAX = (BPT + 1) * 128


def _edge_pass():
    mesh = plsc.VectorSubcoreMesh(core_axis_name="c", subcore_axis_name="s")

    @functools.partial(
        pl.kernel,
        mesh=mesh,
        compiler_params=pltpu.CompilerParams(needs_layout_passes=False),
        out_type=jax.ShapeDtypeStruct((2, NW, NPAD), jnp.float32),
        scratch_types=[
            pltpu.VMEM((N_NODES,), jnp.float32),
            pltpu.VMEM((N_NODES,), jnp.float32),
            pltpu.VMEM((N_NODES,), jnp.float32),
            pltpu.VMEM((2, EMAX), jnp.int32),
            pltpu.VMEM((NPAD,), jnp.float32),
            pltpu.VMEM((NPAD,), jnp.float32),
            pltpu.SemaphoreType.DMA,
        ],
    )
    def edge_pass(px_hbm, py_hbm, pz_hbm, ei_hbm, out_hbm,
                  px_v, py_v, pz_v, rc_v, s_v, d_v, sem):
        wid = lax.axis_index("s") * NC + lax.axis_index("c")
        extra = jnp.where(wid < REM, 1, 0)
        base = (BPT * wid + jnp.minimum(wid, REM)) * 128
        cp_px = pltpu.async_copy(px_hbm, px_v, sem)
        cp_py = pltpu.async_copy(py_hbm, py_v, sem)
        cp_pz = pltpu.async_copy(pz_hbm, pz_v, sem)

        @pl.when(wid < REM)
        def _():
            pltpu.sync_copy(ei_hbm.at[:, pl.ds(base, (BPT + 1) * 128)], rc_v)

        @pl.when(wid >= REM)
        def _():
            pltpu.sync_copy(ei_hbm.at[:, pl.ds(base, BPT * 128)],
                            rc_v.at[:, pl.ds(0, BPT * 128)])

        zf = jnp.zeros((L,), jnp.float32)

        @plsc.parallel_loop(0, NPAD // L, 1, unroll=8)
        def _(i):
            s_v[pl.ds(i * L, L)] = zf
            d_v[pl.ds(i * L, L)] = zf

        cp_px.wait()
        cp_py.wait()
        cp_pz.wait()
        ones = jnp.ones((L,), jnp.float32)

        @plsc.parallel_loop(0, (BPT + extra) * (128 // L), 1, unroll=16)
        def _(i):
            r = rc_v[0, pl.ds(i * L, L)]
            c = rc_v[1, pl.ds(i * L, L)]
            m = r != c
            dx = plsc.load_gather(px_v, [r]) - plsc.load_gather(px_v, [c])
            dy = plsc.load_gather(py_v, [r]) - plsc.load_gather(py_v, [c])
            dz = plsc.load_gather(pz_v, [r]) - plsc.load_gather(pz_v, [c])
            dsq = jnp.maximum(dx * dx + dy * dy + dz * dz, 1e-12)
            yb = jnp.int32(0x5F3759DF) - (plsc.bitcast(dsq, jnp.int32) >> 1)
            y = plsc.bitcast(yb, jnp.float32)
            hx = 0.5 * dsq
            y = y * (1.5 - hx * y * y)
            y = y * (1.5 - hx * y * y)
            plsc.addupdate_scatter(s_v, [r], y, mask=m)
            plsc.addupdate_scatter(d_v, [r], ones, mask=m)

        pltpu.sync_copy(s_v, out_hbm.at[0, wid])
        pltpu.sync_copy(d_v, out_hbm.at[1, wid])

    return edge_pass


_BN = 2560


def _tc_body(x_ref, w_ref, p_ref, o_ref):
    acc = jnp.dot(x_ref[...], w_ref[...], preferred_element_type=jnp.float32)
    s = jnp.sum(p_ref[0], axis=0)
    deg = jnp.sum(p_ref[1], axis=0)
    scale = (1.0 + s) / (deg + 1.0 + 1e-16)
    o_ref[...] = acc * scale[:, None]


@jax.jit
def kernel(x, pos, edge_index, weight, attention):
    del attention
    partials = _edge_pass()(pos[:, 0], pos[:, 1], pos[:, 2], edge_index)

    return pl.pallas_call(
        _tc_body,
        grid=(NPAD // _BN,),
        in_specs=[
            pl.BlockSpec((_BN, 128), lambda i: (i, 0)),
            pl.BlockSpec((128, 128), lambda i: (0, 0)),
            pl.BlockSpec((2, NW, _BN), lambda i: (0, 0, i)),
        ],
        out_specs=pl.BlockSpec((_BN, 128), lambda i: (i, 0)),
        out_shape=jax.ShapeDtypeStruct((N_NODES, 128), jnp.float32),
    )(x, weight[0], partials)


# speedup vs baseline: 1.0959x; 1.0761x over previous
import functools

import jax
import jax.numpy as jnp
from jax import lax
from jax.experimental import pallas as pl
from jax.experimental.pallas import tpu as pltpu
from jax.experimental.pallas import tpu_sc as plsc

N_NODES = 10000
N_EDGES = 320000
NC, NS, L = 2, 16, 16
NW = NC * NS
NPAD = 10240
NBLK = N_EDGES // 128
BPT = NBLK // NW
REM = NBLK - BPT * NW
EMAX = (BPT + 1) * 128


def _edge_pass():
    mesh = plsc.VectorSubcoreMesh(core_axis_name="c", subcore_axis_name="s")

    @functools.partial(
        pl.kernel,
        mesh=mesh,
        compiler_params=pltpu.CompilerParams(needs_layout_passes=False),
        out_type=jax.ShapeDtypeStruct((2, NW, NPAD), jnp.float32),
        scratch_types=[
            pltpu.VMEM((N_NODES,), jnp.float32),
            pltpu.VMEM((N_NODES,), jnp.float32),
            pltpu.VMEM((N_NODES,), jnp.float32),
            pltpu.VMEM((2, EMAX), jnp.int32),
            pltpu.VMEM((NPAD,), jnp.float32),
            pltpu.VMEM((NPAD,), jnp.float32),
            pltpu.SemaphoreType.DMA,
        ],
    )
    def edge_pass(px_hbm, py_hbm, pz_hbm, ei_hbm, out_hbm,
                  px_v, py_v, pz_v, rc_v, s_v, d_v, sem):
        wid = lax.axis_index("s") * NC + lax.axis_index("c")
        extra = jnp.where(wid < REM, 1, 0)
        base = (BPT * wid + jnp.minimum(wid, REM)) * 128
        cp_px = pltpu.async_copy(px_hbm, px_v, sem)
        cp_py = pltpu.async_copy(py_hbm, py_v, sem)
        cp_pz = pltpu.async_copy(pz_hbm, pz_v, sem)

        @pl.when(wid < REM)
        def _():
            pltpu.sync_copy(ei_hbm.at[:, pl.ds(base, (BPT + 1) * 128)], rc_v)

        @pl.when(wid >= REM)
        def _():
            pltpu.sync_copy(ei_hbm.at[:, pl.ds(base, BPT * 128)],
                            rc_v.at[:, pl.ds(0, BPT * 128)])

        zf = jnp.zeros((L,), jnp.float32)

        @plsc.parallel_loop(0, NPAD // L, 1, unroll=8)
        def _(i):
            s_v[pl.ds(i * L, L)] = zf
            d_v[pl.ds(i * L, L)] = zf

        cp_px.wait()
        cp_py.wait()
        cp_pz.wait()
        ones = jnp.ones((L,), jnp.float32)

        @plsc.parallel_loop(0, (BPT + extra) * (128 // L), 1, unroll=8)
        def _(i):
            r = rc_v[0, pl.ds(i * L, L)]
            c = rc_v[1, pl.ds(i * L, L)]
            m = r != c
            dx = plsc.load_gather(px_v, [r]) - plsc.load_gather(px_v, [c])
            dy = plsc.load_gather(py_v, [r]) - plsc.load_gather(py_v, [c])
            dz = plsc.load_gather(pz_v, [r]) - plsc.load_gather(pz_v, [c])
            dsq = jnp.maximum(dx * dx + dy * dy + dz * dz, 1e-12)
            yb = jnp.int32(0x5F3759DF) - (plsc.bitcast(dsq, jnp.int32) >> 1)
            y = plsc.bitcast(yb, jnp.float32)
            hx = 0.5 * dsq
            y = y * (1.5 - hx * y * y)
            y = y * (1.5 - hx * y * y)
            plsc.addupdate_scatter(s_v, [r], y, mask=m)
            plsc.addupdate_scatter(d_v, [r], ones, mask=m)

        pltpu.sync_copy(s_v, out_hbm.at[0, wid])
        pltpu.sync_copy(d_v, out_hbm.at[1, wid])

    return edge_pass


_BN = 2560


def _tc_body(x_ref, w_ref, p_ref, o_ref):
    acc = jnp.dot(x_ref[...], w_ref[...], preferred_element_type=jnp.float32)
    s = jnp.sum(p_ref[0], axis=0)
    deg = jnp.sum(p_ref[1], axis=0)
    scale = (1.0 + s) / (deg + 1.0 + 1e-16)
    o_ref[...] = acc * scale[:, None]


@jax.jit
def kernel(x, pos, edge_index, weight, attention):
    del attention
    partials = _edge_pass()(pos[:, 0], pos[:, 1], pos[:, 2], edge_index)

    return pl.pallas_call(
        _tc_body,
        grid=(NPAD // _BN,),
        in_specs=[
            pl.BlockSpec((_BN, 128), lambda i: (i, 0)),
            pl.BlockSpec((128, 128), lambda i: (0, 0)),
            pl.BlockSpec((2, NW, _BN), lambda i: (0, 0, i)),
        ],
        out_specs=pl.BlockSpec((_BN, 128), lambda i: (i, 0)),
        out_shape=jax.ShapeDtypeStruct((N_NODES, 128), jnp.float32),
    )(x, weight[0], partials)
